# initial kernel scaffold (unmeasured)
import jax
import jax.numpy as jnp
from jax import lax
from jax.experimental import pallas as pl
from jax.experimental.pallas import tpu as pltpu


def kernel(
    x,
):
    def body(*refs):
        pass

    out_shape = jax.ShapeDtypeStruct(..., jnp.float32)
    return pl.pallas_call(body, out_shape=out_shape)(...)


# baseline (device time: 577438 ns/iter reference)
import jax
import jax.numpy as jnp
from jax import lax
from jax.experimental import pallas as pl
from jax.experimental.pallas import tpu as pltpu

N_DEV = 4
M = 4096
N = 4096
CHUNK = N // N_DEV


def kernel(x):
    def body(x_ref, out_ref, comm_ref, stage_ref, send_sems, recv_sems,
             copy_sem, credit_sem):
        p = lax.axis_index("i")
        left = lax.rem(p + 3, N_DEV)
        right = lax.rem(p + 1, N_DEV)

        barrier_sem = pltpu.get_barrier_semaphore()
        for nbr in (left, right):
            pl.semaphore_signal(
                barrier_sem, inc=1,
                device_id=(nbr,), device_id_type=pl.DeviceIdType.MESH,
            )
        pl.semaphore_wait(barrier_sem, 2)

        c0 = lax.rem(p + 3, N_DEV)
        cp = pltpu.make_async_copy(
            x_ref.at[0, :, pl.ds(c0 * CHUNK, CHUNK)], comm_ref.at[0], copy_sem
        )
        cp.start()
        cp.wait()

        for h in range(N_DEV - 1):
            ss, rs = h % 2, (h + 1) % 2
            if h == 2:
                pl.semaphore_wait(credit_sem, 1)
            rdma = pltpu.make_async_remote_copy(
                src_ref=comm_ref.at[ss],
                dst_ref=comm_ref.at[rs],
                send_sem=send_sems.at[ss],
                recv_sem=recv_sems.at[rs],
                device_id=(right,),
                device_id_type=pl.DeviceIdType.MESH,
            )
            rdma.start()
            c = lax.rem(p + 6 - h, N_DEV)
            cp = pltpu.make_async_copy(
                x_ref.at[0, :, pl.ds(c * CHUNK, CHUNK)], stage_ref, copy_sem
            )
            cp.start()
            cp.wait()
            rdma.wait()
            comm_ref[rs] = comm_ref[rs] + stage_ref[...]
            if h == 0:
                pl.semaphore_signal(
                    credit_sem, inc=1,
                    device_id=(left,), device_id_type=pl.DeviceIdType.MESH,
                )

        cp = pltpu.make_async_copy(comm_ref.at[1], out_ref, copy_sem)
        cp.start()
        cp.wait()

    return pl.pallas_call(
        body,
        out_shape=jax.ShapeDtypeStruct((M, CHUNK), jnp.float32),
        in_specs=[pl.BlockSpec(memory_space=pltpu.MemorySpace.HBM)],
        out_specs=pl.BlockSpec(memory_space=pltpu.MemorySpace.HBM),
        scratch_shapes=[
            pltpu.VMEM((2, M, CHUNK), jnp.float32),
            pltpu.VMEM((M, CHUNK), jnp.float32),
            pltpu.SemaphoreType.DMA((2,)),
            pltpu.SemaphoreType.DMA((2,)),
            pltpu.SemaphoreType.DMA,
            pltpu.SemaphoreType.REGULAR,
        ],
        compiler_params=pltpu.CompilerParams(
            collective_id=0,
            vmem_limit_bytes=100 * 1024 * 1024,
        ),
    )(x)


# device time: 307575 ns/iter; 1.8774x vs baseline; 1.8774x over previous
import jax
import jax.numpy as jnp
from jax import lax
from jax.experimental import pallas as pl
from jax.experimental.pallas import tpu as pltpu

N_DEV = 4
M = 4096
N = 4096
CHUNK = N // N_DEV
HALF = CHUNK // 2


def kernel(x):
    def body(x_ref, out_ref, comm_cw, comm_ccw, stage_ref,
             send_cw, recv_cw, send_ccw, recv_ccw,
             copy_sems, credit_cw, credit_ccw):
        p = lax.axis_index("i")
        left = lax.rem(p + 3, N_DEV)
        right = lax.rem(p + 1, N_DEV)

        barrier_sem = pltpu.get_barrier_semaphore()
        for nbr in (left, right):
            pl.semaphore_signal(
                barrier_sem, inc=1,
                device_id=(nbr,), device_id_type=pl.DeviceIdType.MESH,
            )
        pl.semaphore_wait(barrier_sem, 2)

        c0_cw = lax.rem(p + 3, N_DEV)
        c0_ccw = lax.rem(p + 1, N_DEV)
        ld_cw = pltpu.make_async_copy(
            x_ref.at[0, :, pl.ds(c0_cw * CHUNK, HALF)],
            comm_cw.at[0], copy_sems.at[0],
        )
        ld_ccw = pltpu.make_async_copy(
            x_ref.at[0, :, pl.ds(c0_ccw * CHUNK + HALF, HALF)],
            comm_ccw.at[0], copy_sems.at[1],
        )
        ld_cw.start()
        ld_ccw.start()
        ld_cw.wait()
        ld_ccw.wait()

        for h in range(N_DEV - 1):
            ss, rs = h % 2, (h + 1) % 2
            if h == 2:
                pl.semaphore_wait(credit_cw, 1)
                pl.semaphore_wait(credit_ccw, 1)
            rdma_cw = pltpu.make_async_remote_copy(
                src_ref=comm_cw.at[ss], dst_ref=comm_cw.at[rs],
                send_sem=send_cw.at[ss], recv_sem=recv_cw.at[rs],
                device_id=(right,), device_id_type=pl.DeviceIdType.MESH,
            )
            rdma_ccw = pltpu.make_async_remote_copy(
                src_ref=comm_ccw.at[ss], dst_ref=comm_ccw.at[rs],
                send_sem=send_ccw.at[ss], recv_sem=recv_ccw.at[rs],
                device_id=(left,), device_id_type=pl.DeviceIdType.MESH,
            )
            rdma_cw.start()
            rdma_ccw.start()

            rc_cw = lax.rem(p + 6 - h, N_DEV)
            rc_ccw = lax.rem(p + 2 + h, N_DEV)
            st_cw = pltpu.make_async_copy(
                x_ref.at[0, :, pl.ds(rc_cw * CHUNK, HALF)],
                stage_ref.at[0], copy_sems.at[0],
            )
            st_ccw = pltpu.make_async_copy(
                x_ref.at[0, :, pl.ds(rc_ccw * CHUNK + HALF, HALF)],
                stage_ref.at[1], copy_sems.at[1],
            )
            st_cw.start()
            st_ccw.start()

            st_cw.wait()
            rdma_cw.wait()
            comm_cw[rs] = comm_cw[rs] + stage_ref[0]
            st_ccw.wait()
            rdma_ccw.wait()
            comm_ccw[rs] = comm_ccw[rs] + stage_ref[1]

            if h == 0:
                pl.semaphore_signal(
                    credit_cw, inc=1,
                    device_id=(left,), device_id_type=pl.DeviceIdType.MESH,
                )
                pl.semaphore_signal(
                    credit_ccw, inc=1,
                    device_id=(right,), device_id_type=pl.DeviceIdType.MESH,
                )

        out_cw = pltpu.make_async_copy(
            comm_cw.at[1], out_ref.at[:, pl.ds(0, HALF)], copy_sems.at[0]
        )
        out_ccw = pltpu.make_async_copy(
            comm_ccw.at[1], out_ref.at[:, pl.ds(HALF, HALF)], copy_sems.at[1]
        )
        out_cw.start()
        out_ccw.start()
        out_cw.wait()
        out_ccw.wait()

    return pl.pallas_call(
        body,
        out_shape=jax.ShapeDtypeStruct((M, CHUNK), jnp.float32),
        in_specs=[pl.BlockSpec(memory_space=pltpu.MemorySpace.HBM)],
        out_specs=pl.BlockSpec(memory_space=pltpu.MemorySpace.HBM),
        scratch_shapes=[
            pltpu.VMEM((2, M, HALF), jnp.float32),
            pltpu.VMEM((2, M, HALF), jnp.float32),
            pltpu.VMEM((2, M, HALF), jnp.float32),
            pltpu.SemaphoreType.DMA((2,)),
            pltpu.SemaphoreType.DMA((2,)),
            pltpu.SemaphoreType.DMA((2,)),
            pltpu.SemaphoreType.DMA((2,)),
            pltpu.SemaphoreType.DMA((2,)),
            pltpu.SemaphoreType.REGULAR,
            pltpu.SemaphoreType.REGULAR,
        ],
        compiler_params=pltpu.CompilerParams(
            collective_id=0,
            vmem_limit_bytes=100 * 1024 * 1024,
        ),
    )(x)


# device time: 295201 ns/iter; 1.9561x vs baseline; 1.0419x over previous
import jax
import jax.numpy as jnp
from jax import lax
from jax.experimental import pallas as pl
from jax.experimental.pallas import tpu as pltpu

N_DEV = 4
M = 4096
N = 4096
CHUNK = N // N_DEV
HALF = CHUNK // 2
NSUB = 2
SUB = HALF // NSUB


def kernel(x):
    def body(x_ref, out_ref, comm_cw, comm_ccw, stage_ref,
             send_cw, recv_cw, send_ccw, recv_ccw,
             copy_sems, credit_cw, credit_ccw):
        p = lax.axis_index("i")
        left = lax.rem(p + 3, N_DEV)
        right = lax.rem(p + 1, N_DEV)

        rings = [
            dict(idx=0, comm=comm_cw, snd=send_cw, rcv=recv_cw,
                 tgt=right, src=left, off=0, credit=credit_cw,
                 c0=lax.rem(p + 3, N_DEV),
                 rc=lambda h: lax.rem(p + 6 - h, N_DEV)),
            dict(idx=1, comm=comm_ccw, snd=send_ccw, rcv=recv_ccw,
                 tgt=left, src=right, off=HALF, credit=credit_ccw,
                 c0=lax.rem(p + 1, N_DEV),
                 rc=lambda h: lax.rem(p + 2 + h, N_DEV)),
        ]

        def stage_load(r, chunk_idx, b):
            return pltpu.make_async_copy(
                x_ref.at[0, :, pl.ds(chunk_idx * CHUNK + r["off"] + b * SUB, SUB)],
                stage_ref.at[r["idx"], b],
                copy_sems.at[r["idx"] * NSUB + b],
            )

        def hop_rdma(r, ss, rs, b):
            return pltpu.make_async_remote_copy(
                src_ref=r["comm"].at[ss, b],
                dst_ref=r["comm"].at[rs, b],
                send_sem=r["snd"].at[ss * NSUB + b],
                recv_sem=r["rcv"].at[rs * NSUB + b],
                device_id=(r["tgt"],),
                device_id_type=pl.DeviceIdType.MESH,
            )

        def out_copy(r, rs, b):
            return pltpu.make_async_copy(
                r["comm"].at[rs, b],
                out_ref.at[:, pl.ds(r["off"] + b * SUB, SUB)],
                copy_sems.at[r["idx"] * NSUB + b],
            )

        barrier_sem = pltpu.get_barrier_semaphore()
        for nbr in (left, right):
            pl.semaphore_signal(
                barrier_sem, inc=1,
                device_id=(nbr,), device_id_type=pl.DeviceIdType.MESH,
            )
        pl.semaphore_wait(barrier_sem, 2)

        loads = {}
        for r in rings:
            for b in range(NSUB):
                cp = pltpu.make_async_copy(
                    x_ref.at[0, :, pl.ds(r["c0"] * CHUNK + r["off"] + b * SUB, SUB)],
                    r["comm"].at[0, b],
                    copy_sems.at[r["idx"] * NSUB + b],
                )
                cp.start()
                loads[(r["idx"], b)] = cp
        for b in range(NSUB):
            for r in rings:
                loads[(r["idx"], b)].wait()
                hop_rdma(r, 0, 1, b).start()

        for h in range(N_DEV - 1):
            ss, rs = h % 2, (h + 1) % 2
            st = {}
            for r in rings:
                for b in range(NSUB):
                    cp = stage_load(r, r["rc"](h), b)
                    cp.start()
                    st[(r["idx"], b)] = cp

            for b in range(NSUB):
                for r in rings:
                    i = r["idx"]
                    hop_rdma(r, ss, rs, b).wait_recv()
                    st[(i, b)].wait()
                    r["comm"][rs, b] = r["comm"][rs, b] + stage_ref[i, b]
                    if h == 0 and b == NSUB - 1:
                        pl.semaphore_signal(
                            r["credit"], inc=1,
                            device_id=(r["src"],),
                            device_id_type=pl.DeviceIdType.MESH,
                        )
                    if h < N_DEV - 2:
                        if h + 1 == 2:
                            if b == 0:
                                pl.semaphore_wait(r["credit"], 1)
                            hop_rdma(r, rs, ss, b).wait_send()
                        hop_rdma(r, rs, ss, b).start()
                    else:
                        out_copy(r, rs, b).start()

            if h == N_DEV - 2:
                for r in rings:
                    for b in range(NSUB):
                        hop_rdma(r, 1, 0, b).wait_send()
                        hop_rdma(r, 0, 1, b).wait_send()
                        out_copy(r, rs, b).wait()

    return pl.pallas_call(
        body,
        out_shape=jax.ShapeDtypeStruct((M, CHUNK), jnp.float32),
        in_specs=[pl.BlockSpec(memory_space=pltpu.MemorySpace.HBM)],
        out_specs=pl.BlockSpec(memory_space=pltpu.MemorySpace.HBM),
        scratch_shapes=[
            pltpu.VMEM((2, NSUB, M, SUB), jnp.float32),
            pltpu.VMEM((2, NSUB, M, SUB), jnp.float32),
            pltpu.VMEM((2, NSUB, M, SUB), jnp.float32),
            pltpu.SemaphoreType.DMA((2 * NSUB,)),
            pltpu.SemaphoreType.DMA((2 * NSUB,)),
            pltpu.SemaphoreType.DMA((2 * NSUB,)),
            pltpu.SemaphoreType.DMA((2 * NSUB,)),
            pltpu.SemaphoreType.DMA((2 * NSUB,)),
            pltpu.SemaphoreType.REGULAR,
            pltpu.SemaphoreType.REGULAR,
        ],
        compiler_params=pltpu.CompilerParams(
            collective_id=0,
            vmem_limit_bytes=100 * 1024 * 1024,
        ),
    )(x)


# device time: 292428 ns/iter; 1.9746x vs baseline; 1.0095x over previous
import jax
import jax.numpy as jnp
from jax import lax
from jax.experimental import pallas as pl
from jax.experimental.pallas import tpu as pltpu

N_DEV = 4
M = 4096
N = 4096
CHUNK = N // N_DEV
HALF = CHUNK // 2
NSUB = 4
SUB = HALF // NSUB


def kernel(x):
    def body(x_ref, out_ref, comm_cw, comm_ccw, stage_ref,
             send_cw, recv_cw, send_ccw, recv_ccw,
             copy_sems, credit_cw, credit_ccw):
        p = lax.axis_index("i")
        left = lax.rem(p + 3, N_DEV)
        right = lax.rem(p + 1, N_DEV)

        rings = [
            dict(idx=0, comm=comm_cw, snd=send_cw, rcv=recv_cw,
                 tgt=right, src=left, off=0, credit=credit_cw,
                 c0=lax.rem(p + 3, N_DEV),
                 rc=lambda h: lax.rem(p + 6 - h, N_DEV)),
            dict(idx=1, comm=comm_ccw, snd=send_ccw, rcv=recv_ccw,
                 tgt=left, src=right, off=HALF, credit=credit_ccw,
                 c0=lax.rem(p + 1, N_DEV),
                 rc=lambda h: lax.rem(p + 2 + h, N_DEV)),
        ]

        def stage_load(r, chunk_idx, b):
            return pltpu.make_async_copy(
                x_ref.at[0, :, pl.ds(chunk_idx * CHUNK + r["off"] + b * SUB, SUB)],
                stage_ref.at[r["idx"], b],
                copy_sems.at[r["idx"] * NSUB + b],
            )

        def hop_rdma(r, ss, rs, b):
            return pltpu.make_async_remote_copy(
                src_ref=r["comm"].at[ss, b],
                dst_ref=r["comm"].at[rs, b],
                send_sem=r["snd"].at[ss * NSUB + b],
                recv_sem=r["rcv"].at[rs * NSUB + b],
                device_id=(r["tgt"],),
                device_id_type=pl.DeviceIdType.MESH,
            )

        def out_copy(r, rs, b):
            return pltpu.make_async_copy(
                r["comm"].at[rs, b],
                out_ref.at[:, pl.ds(r["off"] + b * SUB, SUB)],
                copy_sems.at[r["idx"] * NSUB + b],
            )

        barrier_sem = pltpu.get_barrier_semaphore()
        for nbr in (left, right):
            pl.semaphore_signal(
                barrier_sem, inc=1,
                device_id=(nbr,), device_id_type=pl.DeviceIdType.MESH,
            )
        pl.semaphore_wait(barrier_sem, 2)

        loads = {}
        for r in rings:
            for b in range(NSUB):
                cp = pltpu.make_async_copy(
                    x_ref.at[0, :, pl.ds(r["c0"] * CHUNK + r["off"] + b * SUB, SUB)],
                    r["comm"].at[0, b],
                    copy_sems.at[r["idx"] * NSUB + b],
                )
                cp.start()
                loads[(r["idx"], b)] = cp
        for b in range(NSUB):
            for r in rings:
                loads[(r["idx"], b)].wait()
                hop_rdma(r, 0, 1, b).start()

        for h in range(N_DEV - 1):
            ss, rs = h % 2, (h + 1) % 2
            st = {}
            for r in rings:
                for b in range(NSUB):
                    cp = stage_load(r, r["rc"](h), b)
                    cp.start()
                    st[(r["idx"], b)] = cp

            for b in range(NSUB):
                for r in rings:
                    i = r["idx"]
                    hop_rdma(r, ss, rs, b).wait_recv()
                    st[(i, b)].wait()
                    r["comm"][rs, b] = r["comm"][rs, b] + stage_ref[i, b]
                    if h == 0 and b == NSUB - 1:
                        pl.semaphore_signal(
                            r["credit"], inc=1,
                            device_id=(r["src"],),
                            device_id_type=pl.DeviceIdType.MESH,
                        )
                    if h < N_DEV - 2:
                        if h + 1 == 2:
                            if b == 0:
                                pl.semaphore_wait(r["credit"], 1)
                            hop_rdma(r, rs, ss, b).wait_send()
                        hop_rdma(r, rs, ss, b).start()
                    else:
                        out_copy(r, rs, b).start()

            if h == N_DEV - 2:
                for r in rings:
                    for b in range(NSUB):
                        hop_rdma(r, 1, 0, b).wait_send()
                        hop_rdma(r, 0, 1, b).wait_send()
                        out_copy(r, rs, b).wait()

    return pl.pallas_call(
        body,
        out_shape=jax.ShapeDtypeStruct((M, CHUNK), jnp.float32),
        in_specs=[pl.BlockSpec(memory_space=pltpu.MemorySpace.HBM)],
        out_specs=pl.BlockSpec(memory_space=pltpu.MemorySpace.HBM),
        scratch_shapes=[
            pltpu.VMEM((2, NSUB, M, SUB), jnp.float32),
            pltpu.VMEM((2, NSUB, M, SUB), jnp.float32),
            pltpu.VMEM((2, NSUB, M, SUB), jnp.float32),
            pltpu.SemaphoreType.DMA((2 * NSUB,)),
            pltpu.SemaphoreType.DMA((2 * NSUB,)),
            pltpu.SemaphoreType.DMA((2 * NSUB,)),
            pltpu.SemaphoreType.DMA((2 * NSUB,)),
            pltpu.SemaphoreType.DMA((2 * NSUB,)),
            pltpu.SemaphoreType.REGULAR,
            pltpu.SemaphoreType.REGULAR,
        ],
        compiler_params=pltpu.CompilerParams(
            collective_id=0,
            vmem_limit_bytes=100 * 1024 * 1024,
        ),
    )(x)


# device time: 292335 ns/iter; 1.9753x vs baseline; 1.0003x over previous
import jax
import jax.numpy as jnp
from jax import lax
from jax.experimental import pallas as pl
from jax.experimental.pallas import tpu as pltpu

N_DEV = 4
M = 4096
N = 4096
CHUNK = N // N_DEV
HALF = CHUNK // 2
NSUB = 4
SUB = HALF // NSUB


def kernel(x):
    def body(x_ref, out_ref, comm_cw, comm_ccw, stage_ref,
             send_cw, recv_cw, send_ccw, recv_ccw,
             copy_sems, credit_cw, credit_ccw):
        p = lax.axis_index("i")
        left = lax.rem(p + 3, N_DEV)
        right = lax.rem(p + 1, N_DEV)

        rings = [
            dict(idx=0, comm=comm_cw, snd=send_cw, rcv=recv_cw,
                 tgt=right, src=left, off=0, credit=credit_cw,
                 c0=lax.rem(p + 3, N_DEV),
                 rc=lambda h: lax.rem(p + 6 - h, N_DEV)),
            dict(idx=1, comm=comm_ccw, snd=send_ccw, rcv=recv_ccw,
                 tgt=left, src=right, off=HALF, credit=credit_ccw,
                 c0=lax.rem(p + 1, N_DEV),
                 rc=lambda h: lax.rem(p + 2 + h, N_DEV)),
        ]

        def stage_load(r, chunk_idx, b):
            return pltpu.make_async_copy(
                x_ref.at[0, :, pl.ds(chunk_idx * CHUNK + r["off"] + b * SUB, SUB)],
                stage_ref.at[r["idx"], b],
                copy_sems.at[r["idx"] * NSUB + b],
            )

        def hop_rdma(r, ss, rs, b):
            return pltpu.make_async_remote_copy(
                src_ref=r["comm"].at[ss, b],
                dst_ref=r["comm"].at[rs, b],
                send_sem=r["snd"].at[ss * NSUB + b],
                recv_sem=r["rcv"].at[rs * NSUB + b],
                device_id=(r["tgt"],),
                device_id_type=pl.DeviceIdType.MESH,
            )

        def out_copy(r, rs, b):
            return pltpu.make_async_copy(
                r["comm"].at[rs, b],
                out_ref.at[:, pl.ds(r["off"] + b * SUB, SUB)],
                copy_sems.at[r["idx"] * NSUB + b],
            )

        loads = {}
        for r in rings:
            for b in range(NSUB):
                cp = pltpu.make_async_copy(
                    x_ref.at[0, :, pl.ds(r["c0"] * CHUNK + r["off"] + b * SUB, SUB)],
                    r["comm"].at[0, b],
                    copy_sems.at[r["idx"] * NSUB + b],
                )
                cp.start()
                loads[(r["idx"], b)] = cp

        barrier_sem = pltpu.get_barrier_semaphore()
        for nbr in (left, right):
            pl.semaphore_signal(
                barrier_sem, inc=1,
                device_id=(nbr,), device_id_type=pl.DeviceIdType.MESH,
            )
        pl.semaphore_wait(barrier_sem, 2)

        for b in range(NSUB):
            for r in rings:
                loads[(r["idx"], b)].wait()
                hop_rdma(r, 0, 1, b).start()

        for h in range(N_DEV - 1):
            ss, rs = h % 2, (h + 1) % 2
            st = {}
            for r in rings:
                for b in range(NSUB):
                    cp = stage_load(r, r["rc"](h), b)
                    cp.start()
                    st[(r["idx"], b)] = cp

            for b in range(NSUB):
                for r in rings:
                    i = r["idx"]
                    hop_rdma(r, ss, rs, b).wait_recv()
                    st[(i, b)].wait()
                    r["comm"][rs, b] = r["comm"][rs, b] + stage_ref[i, b]
                    if h == 0 and b == NSUB - 1:
                        pl.semaphore_signal(
                            r["credit"], inc=1,
                            device_id=(r["src"],),
                            device_id_type=pl.DeviceIdType.MESH,
                        )
                    if h < N_DEV - 2:
                        if h + 1 == 2:
                            if b == 0:
                                pl.semaphore_wait(r["credit"], 1)
                            hop_rdma(r, rs, ss, b).wait_send()
                        hop_rdma(r, rs, ss, b).start()
                    else:
                        out_copy(r, rs, b).start()

            if h == N_DEV - 2:
                for r in rings:
                    for b in range(NSUB):
                        hop_rdma(r, 1, 0, b).wait_send()
                        hop_rdma(r, 0, 1, b).wait_send()
                        out_copy(r, rs, b).wait()

    return pl.pallas_call(
        body,
        out_shape=jax.ShapeDtypeStruct((M, CHUNK), jnp.float32),
        in_specs=[pl.BlockSpec(memory_space=pltpu.MemorySpace.HBM)],
        out_specs=pl.BlockSpec(memory_space=pltpu.MemorySpace.HBM),
        scratch_shapes=[
            pltpu.VMEM((2, NSUB, M, SUB), jnp.float32),
            pltpu.VMEM((2, NSUB, M, SUB), jnp.float32),
            pltpu.VMEM((2, NSUB, M, SUB), jnp.float32),
            pltpu.SemaphoreType.DMA((2 * NSUB,)),
            pltpu.SemaphoreType.DMA((2 * NSUB,)),
            pltpu.SemaphoreType.DMA((2 * NSUB,)),
            pltpu.SemaphoreType.DMA((2 * NSUB,)),
            pltpu.SemaphoreType.DMA((2 * NSUB,)),
            pltpu.SemaphoreType.REGULAR,
            pltpu.SemaphoreType.REGULAR,
        ],
        compiler_params=pltpu.CompilerParams(
            collective_id=0,
            vmem_limit_bytes=100 * 1024 * 1024,
        ),
    )(x)
